# Initial kernel scaffold; baseline (speedup 1.0000x reference)
import jax
import jax.numpy as jnp
from jax.experimental import pallas as pl


def kernel(doy, pe):
    raise NotImplementedError("write your pallas kernel here")


# speedup vs baseline: 5.3423x; 5.3423x over previous
import functools

import jax
import jax.numpy as jnp
from jax import lax
from jax.experimental import pallas as pl
from jax.experimental.pallas import tpu as pltpu
from jax.experimental.pallas import tpu_sc as plsc

D_MODEL = 64
BATCH = 4096
HIST = 200
B_TOTAL = BATCH * HIST

_info = plsc.get_sparse_core_info()
NC = _info.num_cores
NS = _info.num_subcores
NW = NC * NS
B_PER_W = B_TOTAL // NW
BLK = 512
NB = B_PER_W // BLK
D_PAD = 128


def _make_gather():
  mesh = plsc.VectorSubcoreMesh(core_axis_name="c", subcore_axis_name="s")

  @functools.partial(
      pl.kernel,
      mesh=mesh,
      out_type=jax.ShapeDtypeStruct((B_TOTAL, D_PAD), jnp.float32),
      scratch_types=[
          pltpu.VMEM((BLK,), jnp.int32),
          pltpu.VMEM((BLK, D_PAD), jnp.float32),
          pltpu.SemaphoreType.DMA,
      ],
  )
  def gather_kernel(doy_hbm, pe_hbm, out_hbm, idx_v, rows_v, sem):
    wid = lax.axis_index("s") * NC + lax.axis_index("c")

    def body(i, carry):
      base = wid * B_PER_W + i * BLK
      pltpu.sync_copy(doy_hbm.at[pl.ds(base, BLK)], idx_v)
      pltpu.async_copy(pe_hbm.at[idx_v], rows_v, sem).wait()
      pltpu.sync_copy(rows_v, out_hbm.at[pl.ds(base, BLK)])
      return carry

    lax.fori_loop(0, NB, body, 0)

  return gather_kernel


_gather = _make_gather()


def kernel(doy, pe):
  flat_idx = doy.reshape(B_TOTAL)
  pe_pad = jnp.pad(pe, ((0, 0), (0, D_PAD - D_MODEL)))
  out = _gather(flat_idx, pe_pad)
  return out[:, :D_MODEL].reshape(BATCH, HIST, D_MODEL, 1, 1)
